# baseline (speedup 1.0000x reference)
import functools

import jax
import jax.numpy as jnp
from jax import lax
from jax.experimental import pallas as pl
from jax.experimental.pallas import tpu as pltpu
from jax.experimental.pallas import tpu_sc as plsc

R = 64
N = 32768
L = 16
NC = 2
NS = 16
NW = NC * NS
ROWS_PER_W = R // NW
U = 4
NVEC = N // L
NSTEP = NVEC // U

_mesh = plsc.VectorSubcoreMesh(
    core_axis_name="c", subcore_axis_name="s", num_cores=NC, num_subcores=NS
)


_GATHER_DNUMS = lax.GatherDimensionNumbers(
    offset_dims=(), collapsed_slice_dims=(0,), start_index_map=(0,)
)


def _lane_shuffle(x, perm):
    return lax.gather(
        x, perm[:, None], _GATHER_DNUMS, slice_sizes=(1,),
        mode=lax.GatherScatterMode.PROMISE_IN_BOUNDS,
    )


def _merge(va, sa, vb, sb):
    take_b = (vb > va) | ((vb == va) & (sb < sa))
    return jnp.where(take_b, vb, va), jnp.where(take_b, sb, sa)


@functools.partial(
    pl.kernel,
    out_type=jax.ShapeDtypeStruct((NW, L), jnp.int32),
    mesh=_mesh,
    scratch_types=[
        pltpu.VMEM((ROWS_PER_W, N), jnp.float32),
        pltpu.VMEM((L,), jnp.int32),
        pltpu.SemaphoreType.DMA,
    ],
)
def _argmax_sc(x_hbm, out_hbm, rows_v, res_v, sem):
    c = lax.axis_index("c")
    s = lax.axis_index("s")
    wid = c * NS + s
    base = wid * ROWS_PER_W

    pltpu.async_copy(x_hbm.at[pl.ds(base, ROWS_PER_W)], rows_v, sem).wait()

    lane = lax.iota(jnp.int32, L)
    neg_inf = jnp.full((L,), -jnp.inf, dtype=jnp.float32)

    res = jnp.zeros((L,), jnp.int32)
    for r in range(ROWS_PER_W):
        m = [neg_inf] * U
        st = [jnp.full((L,), u, dtype=jnp.int32) for u in range(U)]

        def body(i, carry):
            ms, sts = carry
            ms, sts = list(ms), list(sts)
            off = i * (U * L)
            for u in range(U):
                v = rows_v[r, pl.ds(off + u * L, L)]
                upd = v > ms[u]
                ms[u] = jnp.where(upd, v, ms[u])
                sts[u] = jnp.where(upd, jnp.full((L,), i * U + u, jnp.int32),
                                   sts[u])
            return tuple(ms), tuple(sts)

        (m, st) = lax.fori_loop(0, NSTEP, body, (tuple(m), tuple(st)))
        m, st = list(m), list(st)

        mv, sv = m[0], st[0]
        for u in range(1, U):
            mv, sv = _merge(mv, sv, m[u], st[u])

        bv = mv
        bi = sv * L + lane
        for k in (1, 2, 4, 8):
            perm = lane ^ k
            ov = _lane_shuffle(bv, perm)
            oi = _lane_shuffle(bi, perm)
            take_o = (ov > bv) | ((ov == bv) & (oi < bi))
            bv = jnp.where(take_o, ov, bv)
            bi = jnp.where(take_o, oi, bi)

        res = jnp.where(lane == r, bi, res)

    res_v[...] = res
    pltpu.sync_copy(res_v, out_hbm.at[wid])


def kernel(X):
    out = _argmax_sc(X)
    return out[:, :ROWS_PER_W].reshape(R)


# speedup vs baseline: 2.3034x; 2.3034x over previous
import jax
import jax.numpy as jnp
from jax import lax
from jax.experimental import pallas as pl
from jax.experimental.pallas import tpu as pltpu

R = 64
N = 32768
LANES = 128
BLK = 2048
K = N // BLK
SLICES = BLK // LANES


def _argmax_body(x_ref, o_ref, m_ref, i_ref):
    step = pl.program_id(0)

    @pl.when(step == 0)
    def _():
        m_ref[...] = jnp.full((R, LANES), -jnp.inf, jnp.float32)
        i_ref[...] = jnp.zeros((R, LANES), jnp.int32)

    xb = x_ref[...]
    m = m_ref[...]
    i = i_ref[...]
    for s in range(SLICES):
        sl = xb[:, s * LANES:(s + 1) * LANES]
        gt = sl > m
        m = jnp.where(gt, sl, m)
        i = jnp.where(gt, jnp.full((R, LANES), step * SLICES + s, jnp.int32), i)
    m_ref[...] = m
    i_ref[...] = i

    @pl.when(step == K - 1)
    def _():
        lane = lax.broadcasted_iota(jnp.int32, (R, LANES), 1)
        col = i * LANES + lane
        gmax = jnp.max(m, axis=1, keepdims=True)
        cand = jnp.where(m == gmax, col, jnp.full((R, LANES), N, jnp.int32))
        o_ref[...] = jnp.min(cand, axis=1, keepdims=True)


@jax.jit
def kernel(X):
    out = pl.pallas_call(
        _argmax_body,
        grid=(K,),
        in_specs=[pl.BlockSpec((R, BLK), lambda k: (0, k))],
        out_specs=pl.BlockSpec((R, 1), lambda k: (0, 0)),
        out_shape=jax.ShapeDtypeStruct((R, 1), jnp.int32),
        scratch_shapes=[
            pltpu.VMEM((R, LANES), jnp.float32),
            pltpu.VMEM((R, LANES), jnp.int32),
        ],
    )(X)
    return out.reshape(R)


# speedup vs baseline: 3.1834x; 1.3821x over previous
import jax
import jax.numpy as jnp
from jax import lax
from jax.experimental import pallas as pl
from jax.experimental.pallas import tpu as pltpu

R = 64
N = 32768
LANES = 128
A = 2
W = A * LANES
BLK = 4096
K = N // BLK
SLICES = BLK // LANES


def _argmax_body(x_ref, o_ref, m_ref, i_ref):
    step = pl.program_id(0)

    @pl.when(step == 0)
    def _():
        m_ref[...] = jnp.full((R, W), -jnp.inf, jnp.float32)
        i_ref[...] = jnp.zeros((R, W), jnp.int32)

    m = [m_ref[:, a * LANES:(a + 1) * LANES] for a in range(A)]
    i = [i_ref[:, a * LANES:(a + 1) * LANES] for a in range(A)]
    for s in range(SLICES):
        a = s % A
        sl = x_ref[:, s * LANES:(s + 1) * LANES]
        sid = jnp.full((R, LANES), step * SLICES + s, jnp.int32)
        gt = sl > m[a]
        m[a] = jnp.where(gt, sl, m[a])
        i[a] = jnp.where(gt, sid, i[a])
    for a in range(A):
        m_ref[:, a * LANES:(a + 1) * LANES] = m[a]
        i_ref[:, a * LANES:(a + 1) * LANES] = i[a]

    @pl.when(step == K - 1)
    def _():
        lane = lax.broadcasted_iota(jnp.int32, (R, LANES), 1)
        mv = m[0]
        iv = i[0] * LANES + lane
        for a in range(1, A):
            ov = m[a]
            oi = i[a] * LANES + lane
            take = (ov > mv) | ((ov == mv) & (oi < iv))
            mv = jnp.where(take, ov, mv)
            iv = jnp.where(take, oi, iv)
        gmax = jnp.max(mv, axis=1, keepdims=True)
        cand = jnp.where(mv == gmax, iv, jnp.full((R, LANES), N, jnp.int32))
        o_ref[...] = jnp.min(cand, axis=1, keepdims=True)


@jax.jit
def kernel(X):
    out = pl.pallas_call(
        _argmax_body,
        grid=(K,),
        in_specs=[pl.BlockSpec((R, BLK), lambda k: (0, k))],
        out_specs=pl.BlockSpec((R, 1), lambda k: (0, 0)),
        out_shape=jax.ShapeDtypeStruct((R, 1), jnp.int32),
        scratch_shapes=[
            pltpu.VMEM((R, W), jnp.float32),
            pltpu.VMEM((R, W), jnp.int32),
        ],
    )(X)
    return out.reshape(R)


# speedup vs baseline: 4.2391x; 1.3316x over previous
import jax
import jax.numpy as jnp
from jax import lax
from jax.experimental import pallas as pl
from jax.experimental.pallas import tpu as pltpu

R = 64
N = 32768
LANES = 128
A = 2
W = A * LANES
BLK = 16384
K = N // BLK
SLICES = BLK // LANES


def _argmax_body(x_ref, o_ref, m_ref, i_ref):
    step = pl.program_id(0)

    @pl.when(step == 0)
    def _():
        m_ref[...] = jnp.full((R, W), -jnp.inf, jnp.float32)
        i_ref[...] = jnp.zeros((R, W), jnp.int32)

    m = [m_ref[:, a * LANES:(a + 1) * LANES] for a in range(A)]
    i = [i_ref[:, a * LANES:(a + 1) * LANES] for a in range(A)]
    for s in range(SLICES):
        a = s % A
        sl = x_ref[:, s * LANES:(s + 1) * LANES]
        sid = jnp.full((R, LANES), step * SLICES + s, jnp.int32)
        gt = sl > m[a]
        m[a] = jnp.where(gt, sl, m[a])
        i[a] = jnp.where(gt, sid, i[a])
    for a in range(A):
        m_ref[:, a * LANES:(a + 1) * LANES] = m[a]
        i_ref[:, a * LANES:(a + 1) * LANES] = i[a]

    @pl.when(step == K - 1)
    def _():
        lane = lax.broadcasted_iota(jnp.int32, (R, LANES), 1)
        mv = m[0]
        iv = i[0] * LANES + lane
        for a in range(1, A):
            ov = m[a]
            oi = i[a] * LANES + lane
            take = (ov > mv) | ((ov == mv) & (oi < iv))
            mv = jnp.where(take, ov, mv)
            iv = jnp.where(take, oi, iv)
        gmax = jnp.max(mv, axis=1, keepdims=True)
        cand = jnp.where(mv == gmax, iv, jnp.full((R, LANES), N, jnp.int32))
        o_ref[...] = jnp.min(cand, axis=1, keepdims=True)


@jax.jit
def kernel(X):
    out = pl.pallas_call(
        _argmax_body,
        grid=(K,),
        in_specs=[pl.BlockSpec((R, BLK), lambda k: (0, k))],
        out_specs=pl.BlockSpec((R, 1), lambda k: (0, 0)),
        out_shape=jax.ShapeDtypeStruct((R, 1), jnp.int32),
        scratch_shapes=[
            pltpu.VMEM((R, W), jnp.float32),
            pltpu.VMEM((R, W), jnp.int32),
        ],
    )(X)
    return out.reshape(R)
